# Initial kernel scaffold; baseline (speedup 1.0000x reference)
import jax
import jax.numpy as jnp
from jax.experimental import pallas as pl


def kernel(mu_pred, sigma_pred, edge_index, edge_attr, pos_emb, We1, be1, We2, be2, Wroot1, Wmsg1, bconv1, Wroot2, Wmsg2, bconv2, Wmu, bmu, Wrad, brad):
    raise NotImplementedError("write your pallas kernel here")


# speedup vs baseline: 2.2860x; 2.2860x over previous
import functools

import jax
import jax.numpy as jnp
from jax import lax
from jax.experimental import pallas as pl
from jax.experimental.pallas import tpu as pltpu
from jax.experimental.pallas import tpu_sc as plsc

N = 50000
E = 800000
H = 64
HH = 32

CHUNK = 128
EP = 819200
NCH = EP // CHUNK
CPT = NCH // 16
CNT_CPT = NCH // 32
GRP = 40
NPAD = 50048
STRIPE = NPAD // 16
ZROWS = STRIPE // 4
SGRP = 16
SZROWS = STRIPE // 8

BN = 2000
BE = 3200

_f32 = jnp.float32
_i32 = jnp.int32


def _gate_body(ea_ref, we1_ref, be1_ref, we2_ref, be2_ref, out_ref):
    h = jnp.maximum(
        jnp.dot(ea_ref[...], we1_ref[...], preferred_element_type=_f32)
        + be1_ref[...], 0.0)
    g = jnp.dot(h, we2_ref[...], preferred_element_type=_f32) + be2_ref[...]
    out_ref[0] = g[:, :HH]
    out_ref[1] = g[:, HH:]


def _gate_call(edge_attr, We1, be1, We2, be2):
    return pl.pallas_call(
        _gate_body,
        grid=(E // BE,),
        in_specs=[
            pl.BlockSpec((BE, 16), lambda i: (i, 0)),
            pl.BlockSpec((16, H), lambda i: (0, 0)),
            pl.BlockSpec((1, H), lambda i: (0, 0)),
            pl.BlockSpec((H, H), lambda i: (0, 0)),
            pl.BlockSpec((1, H), lambda i: (0, 0)),
        ],
        out_specs=pl.BlockSpec((2, BE, HH), lambda i: (0, i, 0)),
        out_shape=jax.ShapeDtypeStruct((2, EP, HH), _f32),
    )(edge_attr, We1, be1, We2, be2)


def _node1_body(sig_ref, pos_ref, wmsg_ref, wroot_ref, y0_ref, y1_ref, root_ref):
    pos = pos_ref[...]
    for b, y_ref in ((0, y0_ref), (1, y1_ref)):
        x = jnp.concatenate([sig_ref[b], pos], axis=1)
        y = jnp.dot(x, wmsg_ref[...], preferred_element_type=_f32)
        y_ref[0] = y[:, :HH]
        y_ref[1] = y[:, HH:]
        root_ref[b] = jnp.dot(x, wroot_ref[...], preferred_element_type=_f32)


def _node1_call(sigma, pos_emb, Wmsg1, Wroot1):
    return pl.pallas_call(
        _node1_body,
        grid=(N // BN,),
        in_specs=[
            pl.BlockSpec((2, BN, 1), lambda i: (0, i, 0)),
            pl.BlockSpec((BN, 16), lambda i: (i, 0)),
            pl.BlockSpec((17, H), lambda i: (0, 0)),
            pl.BlockSpec((17, H), lambda i: (0, 0)),
        ],
        out_specs=[
            pl.BlockSpec((2, BN, HH), lambda i: (0, i, 0)),
            pl.BlockSpec((2, BN, HH), lambda i: (0, i, 0)),
            pl.BlockSpec((2, BN, H), lambda i: (0, i, 0)),
        ],
        out_shape=[
            jax.ShapeDtypeStruct((2, N, HH), _f32),
            jax.ShapeDtypeStruct((2, N, HH), _f32),
            jax.ShapeDtypeStruct((2, N, H), _f32),
        ],
    )(sigma, pos_emb, Wmsg1, Wroot1)


def _rcnt_from(cntp):
    cnt = jnp.maximum(cntp[0, :, 0] + cntp[1, :, 0], 1.0)
    return (1.0 / cnt)[:, None]


def _mid_body(root1_ref, a0_ref, a1_ref, cnt_ref, wmsg_ref, wroot_ref, b1_ref,
              y0_ref, y1_ref, root2_ref):
    rcnt = _rcnt_from(cnt_ref[...])
    for b, a_ref, y_ref in ((0, a0_ref, y0_ref), (1, a1_ref, y1_ref)):
        agg = jnp.concatenate([a_ref[0], a_ref[1]], axis=1) * rcnt
        h = jnp.maximum(root1_ref[b] + agg + b1_ref[...], 0.0)
        y = jnp.dot(h, wmsg_ref[...], preferred_element_type=_f32)
        y_ref[0] = y[:, :HH]
        y_ref[1] = y[:, HH:]
        root2_ref[b] = jnp.dot(h, wroot_ref[...], preferred_element_type=_f32)


def _mid_call(root1, agg0, agg1, cntp, Wmsg2, Wroot2, bconv1):
    return pl.pallas_call(
        _mid_body,
        grid=(N // BN,),
        in_specs=[
            pl.BlockSpec((2, BN, H), lambda i: (0, i, 0)),
            pl.BlockSpec((2, BN, HH), lambda i: (0, i, 0)),
            pl.BlockSpec((2, BN, HH), lambda i: (0, i, 0)),
            pl.BlockSpec((2, BN, 16), lambda i: (0, i, 0)),
            pl.BlockSpec((H, H), lambda i: (0, 0)),
            pl.BlockSpec((H, H), lambda i: (0, 0)),
            pl.BlockSpec((1, H), lambda i: (0, 0)),
        ],
        out_specs=[
            pl.BlockSpec((2, BN, HH), lambda i: (0, i, 0)),
            pl.BlockSpec((2, BN, HH), lambda i: (0, i, 0)),
            pl.BlockSpec((2, BN, H), lambda i: (0, i, 0)),
        ],
        out_shape=[
            jax.ShapeDtypeStruct((2, N, HH), _f32),
            jax.ShapeDtypeStruct((2, N, HH), _f32),
            jax.ShapeDtypeStruct((2, N, H), _f32),
        ],
    )(root1, agg0, agg1, cntp, Wmsg2, Wroot2, bconv1)


def _head_body(root2_ref, a0_ref, a1_ref, cnt_ref, mu_ref, wh_ref, bh_ref,
               b2_ref, muo_ref, ro_ref):
    rcnt = _rcnt_from(cnt_ref[...])
    for b, a_ref in ((0, a0_ref), (1, a1_ref)):
        agg = jnp.concatenate([a_ref[0], a_ref[1]], axis=1) * rcnt
        h = jnp.maximum(root2_ref[b] + agg + b2_ref[...], 0.0)
        z = jnp.dot(h, wh_ref[...], preferred_element_type=_f32) + bh_ref[...]
        muo_ref[b] = mu_ref[b] + z[:, 0:1]
        zr = z[:, 1:2]
        ro_ref[b] = jnp.maximum(zr, 0.0) + jnp.log1p(jnp.exp(-jnp.abs(zr)))


def _head_call(root2, agg0, agg1, cntp, mu_pred, Whead, bhead, bconv2):
    return pl.pallas_call(
        _head_body,
        grid=(N // BN,),
        in_specs=[
            pl.BlockSpec((2, BN, H), lambda i: (0, i, 0)),
            pl.BlockSpec((2, BN, HH), lambda i: (0, i, 0)),
            pl.BlockSpec((2, BN, HH), lambda i: (0, i, 0)),
            pl.BlockSpec((2, BN, 16), lambda i: (0, i, 0)),
            pl.BlockSpec((2, BN, 1), lambda i: (0, i, 0)),
            pl.BlockSpec((H, 2), lambda i: (0, 0)),
            pl.BlockSpec((1, 2), lambda i: (0, 0)),
            pl.BlockSpec((1, H), lambda i: (0, 0)),
        ],
        out_specs=[
            pl.BlockSpec((2, BN, 1), lambda i: (0, i, 0)),
            pl.BlockSpec((2, BN, 1), lambda i: (0, i, 0)),
        ],
        out_shape=[
            jax.ShapeDtypeStruct((2, N, 1), _f32),
            jax.ShapeDtypeStruct((2, N, 1), _f32),
        ],
    )(root2, agg0, agg1, cntp, mu_pred, Whead, bhead, bconv2)


@functools.cache
def _mesh():
    return plsc.VectorSubcoreMesh(
        core_axis_name="c", subcore_axis_name="s", num_cores=2, num_subcores=16)


def _fill_zeros(zbuf, rows, cols):
    zero = jnp.zeros((16,), _f32)

    def body(i, _):
        for h in range(cols // 16):
            zbuf[i, pl.ds(h * 16, 16)] = zero
        return 0

    lax.fori_loop(0, rows, body, 0)


def _cnt_kernel(dst_hbm, out_hbm, dstg, ones, cnt_s, zbuf):
    c = lax.axis_index("c")
    s = lax.axis_index("s")

    _fill_zeros(zbuf, ZROWS, 16)
    one = jnp.full((16,), 1.0, _f32)

    def fill_ones(i, _):
        ones[i, pl.ds(0, 16)] = one
        return 0

    lax.fori_loop(0, CHUNK, fill_ones, 0)

    row0 = s * STRIPE
    for q in range(4):
        pltpu.sync_copy(zbuf, cnt_s.at[pl.ds(row0 + q * ZROWS, ZROWS), :])
    plsc.subcore_barrier()

    base = c * (NCH // 2) + s * CNT_CPT
    for g in range(CNT_CPT // GRP):
        pltpu.sync_copy(dst_hbm.at[pl.ds(base + g * GRP, GRP), :], dstg)

        def body(k, _):
            pltpu.sync_copy(ones, cnt_s.at[dstg.at[k]], add=True)
            return 0

        lax.fori_loop(0, GRP, body, 0)

    plsc.subcore_barrier()
    pltpu.sync_copy(cnt_s.at[pl.ds(row0, STRIPE), :],
                    out_hbm.at[c, pl.ds(row0, STRIPE), :])


def _cnt_call(dst_c):
    f = functools.partial(
        pl.kernel,
        out_type=jax.ShapeDtypeStruct((2, NPAD, 16), _f32),
        mesh=_mesh(),
        compiler_params=pltpu.CompilerParams(use_tc_tiling_on_sc=False),
        scratch_types=[
            pltpu.VMEM((GRP, CHUNK), _i32),
            pltpu.VMEM((CHUNK, 16), _f32),
            pltpu.VMEM_SHARED((NPAD, 16), _f32),
            pltpu.VMEM((ZROWS, 16), _f32),
        ],
    )
    return f(_cnt_kernel)(dst_c)


def _seg_kernel(tab0_hbm, tab1_hbm, gate_hbm, src_hbm, dst_hbm,
                out0_hbm, out1_hbm, srcg, dstg, rows, gatev, agg_s, zbuf, sem):
    c = lax.axis_index("c")
    s = lax.axis_index("s")

    _fill_zeros(zbuf, SZROWS, HH)
    row0 = s * STRIPE

    for b, tab_hbm, out_hbm in ((0, tab0_hbm, out0_hbm), (1, tab1_hbm, out1_hbm)):
        for q in range(8):
            pltpu.sync_copy(zbuf, agg_s.at[pl.ds(row0 + q * SZROWS, SZROWS), :])
        plsc.subcore_barrier()

        base = s * CPT
        for g in range(CPT // SGRP):
            g0 = base + g * SGRP
            pltpu.sync_copy(src_hbm.at[pl.ds(g0, SGRP), :], srcg)
            pltpu.sync_copy(dst_hbm.at[pl.ds(g0, SGRP), :], dstg)

            def body(k, _):
                pltpu.async_copy(tab_hbm.at[c].at[srcg.at[k]], rows, sem).wait()
                pltpu.sync_copy(
                    gate_hbm.at[c, pl.ds((g0 + k) * CHUNK, CHUNK), :], gatev)

                def mul(r, _):
                    rows[r, pl.ds(0, 16)] = (
                        rows[r, pl.ds(0, 16)] * gatev[r, pl.ds(0, 16)])
                    rows[r, pl.ds(16, 16)] = (
                        rows[r, pl.ds(16, 16)] * gatev[r, pl.ds(16, 16)])
                    return 0

                lax.fori_loop(0, CHUNK, mul, 0)
                pltpu.sync_copy(rows, agg_s.at[dstg.at[k]], add=True)
                return 0

            lax.fori_loop(0, SGRP, body, 0)

        plsc.subcore_barrier()
        pltpu.sync_copy(agg_s.at[pl.ds(row0, STRIPE), :],
                        out_hbm.at[c, pl.ds(row0, STRIPE), :])
        plsc.subcore_barrier()


def _seg_call(tab0, tab1, gate, src_c, dst_c):
    f = functools.partial(
        pl.kernel,
        out_type=[
            jax.ShapeDtypeStruct((2, NPAD, HH), _f32),
            jax.ShapeDtypeStruct((2, NPAD, HH), _f32),
        ],
        mesh=_mesh(),
        compiler_params=pltpu.CompilerParams(use_tc_tiling_on_sc=False),
        scratch_types=[
            pltpu.VMEM((SGRP, CHUNK), _i32),
            pltpu.VMEM((SGRP, CHUNK), _i32),
            pltpu.VMEM((CHUNK, HH), _f32),
            pltpu.VMEM((CHUNK, HH), _f32),
            pltpu.VMEM_SHARED((NPAD, HH), _f32),
            pltpu.VMEM((SZROWS, HH), _f32),
            pltpu.SemaphoreType.DMA,
        ],
    )
    return f(_seg_kernel)(tab0, tab1, gate, src_c, dst_c)


def kernel(mu_pred, sigma_pred, edge_index, edge_attr, pos_emb, We1, be1, We2,
           be2, Wroot1, Wmsg1, bconv1, Wroot2, Wmsg2, bconv2, Wmu, bmu, Wrad,
           brad):
    src = edge_index[0].astype(_i32)
    dst = edge_index[1].astype(_i32)
    src_c = jnp.reshape(
        jnp.concatenate([src, jnp.zeros((EP - E,), _i32)]), (NCH, CHUNK))
    dst_c = jnp.reshape(
        jnp.concatenate([dst, jnp.full((EP - E,), N, _i32)]), (NCH, CHUNK))

    gate = _gate_call(edge_attr, We1, be1[None, :], We2, be2[None, :])
    cntp = _cnt_call(dst_c)

    y1_b0, y1_b1, root1 = _node1_call(sigma_pred[..., None], pos_emb, Wmsg1, Wroot1)
    agg1_b0, agg1_b1 = _seg_call(y1_b0, y1_b1, gate, src_c, dst_c)

    y2_b0, y2_b1, root2 = _mid_call(
        root1, agg1_b0, agg1_b1, cntp, Wmsg2, Wroot2, bconv1[None, :])
    agg2_b0, agg2_b1 = _seg_call(y2_b0, y2_b1, gate, src_c, dst_c)

    Whead = jnp.concatenate([Wmu, Wrad], axis=1)
    bhead = jnp.concatenate([bmu, brad])[None, :]
    mu_out, r_out = _head_call(
        root2, agg2_b0, agg2_b1, cntp, mu_pred[..., None], Whead, bhead,
        bconv2[None, :])
    return (mu_out[..., 0], r_out[..., 0])


# speedup vs baseline: 3.4589x; 1.5131x over previous
import functools

import jax
import jax.numpy as jnp
from jax import lax
from jax.experimental import pallas as pl
from jax.experimental.pallas import tpu as pltpu
from jax.experimental.pallas import tpu_sc as plsc

N = 50000
E = 800000
H = 64
HH = 32

CHUNK = 128
EP = 819200
NCH = EP // CHUNK
CPT = NCH // 16
CNT_CPT = NCH // 32
GRP = 40
NPAD = 50048
STRIPE = NPAD // 16
ZROWS = STRIPE // 4
SGRP = 16
SZROWS = STRIPE // 8

BN = 2000
BE = 3200

_f32 = jnp.float32
_i32 = jnp.int32


def _gate_body(ea_ref, we1_ref, be1_ref, we2_ref, be2_ref, out_ref):
    h = jnp.maximum(
        jnp.dot(ea_ref[...], we1_ref[...], preferred_element_type=_f32)
        + be1_ref[...], 0.0)
    g = jnp.dot(h, we2_ref[...], preferred_element_type=_f32) + be2_ref[...]
    out_ref[0] = g[:, :HH]
    out_ref[1] = g[:, HH:]


def _gate_call(edge_attr, We1, be1, We2, be2):
    return pl.pallas_call(
        _gate_body,
        grid=(E // BE,),
        in_specs=[
            pl.BlockSpec((BE, 16), lambda i: (i, 0)),
            pl.BlockSpec((16, H), lambda i: (0, 0)),
            pl.BlockSpec((1, H), lambda i: (0, 0)),
            pl.BlockSpec((H, H), lambda i: (0, 0)),
            pl.BlockSpec((1, H), lambda i: (0, 0)),
        ],
        out_specs=pl.BlockSpec((2, BE, HH), lambda i: (0, i, 0)),
        out_shape=jax.ShapeDtypeStruct((2, EP, HH), _f32),
    )(edge_attr, We1, be1, We2, be2)


def _node1_body(sig_ref, pos_ref, wmsg_ref, wroot_ref, y0_ref, y1_ref, root_ref):
    pos = pos_ref[...]
    for b, y_ref in ((0, y0_ref), (1, y1_ref)):
        x = jnp.concatenate([sig_ref[b], pos], axis=1)
        y = jnp.dot(x, wmsg_ref[...], preferred_element_type=_f32)
        y_ref[0] = y[:, :HH]
        y_ref[1] = y[:, HH:]
        root_ref[b] = jnp.dot(x, wroot_ref[...], preferred_element_type=_f32)


def _node1_call(sigma, pos_emb, Wmsg1, Wroot1):
    return pl.pallas_call(
        _node1_body,
        grid=(N // BN,),
        in_specs=[
            pl.BlockSpec((2, BN, 1), lambda i: (0, i, 0)),
            pl.BlockSpec((BN, 16), lambda i: (i, 0)),
            pl.BlockSpec((17, H), lambda i: (0, 0)),
            pl.BlockSpec((17, H), lambda i: (0, 0)),
        ],
        out_specs=[
            pl.BlockSpec((2, BN, HH), lambda i: (0, i, 0)),
            pl.BlockSpec((2, BN, HH), lambda i: (0, i, 0)),
            pl.BlockSpec((2, BN, H), lambda i: (0, i, 0)),
        ],
        out_shape=[
            jax.ShapeDtypeStruct((2, N, HH), _f32),
            jax.ShapeDtypeStruct((2, N, HH), _f32),
            jax.ShapeDtypeStruct((2, N, H), _f32),
        ],
    )(sigma, pos_emb, Wmsg1, Wroot1)


def _rcnt_from(cntp):
    cnt = jnp.maximum(cntp[0, :, 0] + cntp[1, :, 0], 1.0)
    return (1.0 / cnt)[:, None]


def _mid_body(root1_ref, a0_ref, a1_ref, cnt_ref, wmsg_ref, wroot_ref, b1_ref,
              y0_ref, y1_ref, root2_ref):
    rcnt = _rcnt_from(cnt_ref[...])
    for b, a_ref, y_ref in ((0, a0_ref, y0_ref), (1, a1_ref, y1_ref)):
        agg = jnp.concatenate([a_ref[0], a_ref[1]], axis=1) * rcnt
        h = jnp.maximum(root1_ref[b] + agg + b1_ref[...], 0.0)
        y = jnp.dot(h, wmsg_ref[...], preferred_element_type=_f32)
        y_ref[0] = y[:, :HH]
        y_ref[1] = y[:, HH:]
        root2_ref[b] = jnp.dot(h, wroot_ref[...], preferred_element_type=_f32)


def _mid_call(root1, agg0, agg1, cntp, Wmsg2, Wroot2, bconv1):
    return pl.pallas_call(
        _mid_body,
        grid=(N // BN,),
        in_specs=[
            pl.BlockSpec((2, BN, H), lambda i: (0, i, 0)),
            pl.BlockSpec((2, BN, HH), lambda i: (0, i, 0)),
            pl.BlockSpec((2, BN, HH), lambda i: (0, i, 0)),
            pl.BlockSpec((2, BN, 16), lambda i: (0, i, 0)),
            pl.BlockSpec((H, H), lambda i: (0, 0)),
            pl.BlockSpec((H, H), lambda i: (0, 0)),
            pl.BlockSpec((1, H), lambda i: (0, 0)),
        ],
        out_specs=[
            pl.BlockSpec((2, BN, HH), lambda i: (0, i, 0)),
            pl.BlockSpec((2, BN, HH), lambda i: (0, i, 0)),
            pl.BlockSpec((2, BN, H), lambda i: (0, i, 0)),
        ],
        out_shape=[
            jax.ShapeDtypeStruct((2, N, HH), _f32),
            jax.ShapeDtypeStruct((2, N, HH), _f32),
            jax.ShapeDtypeStruct((2, N, H), _f32),
        ],
    )(root1, agg0, agg1, cntp, Wmsg2, Wroot2, bconv1)


def _head_body(root2_ref, a0_ref, a1_ref, cnt_ref, mu_ref, wh_ref, bh_ref,
               b2_ref, muo_ref, ro_ref):
    rcnt = _rcnt_from(cnt_ref[...])
    for b, a_ref in ((0, a0_ref), (1, a1_ref)):
        agg = jnp.concatenate([a_ref[0], a_ref[1]], axis=1) * rcnt
        h = jnp.maximum(root2_ref[b] + agg + b2_ref[...], 0.0)
        z = jnp.dot(h, wh_ref[...], preferred_element_type=_f32) + bh_ref[...]
        muo_ref[b] = mu_ref[b] + z[:, 0:1]
        zr = z[:, 1:2]
        ro_ref[b] = jnp.maximum(zr, 0.0) + jnp.log1p(jnp.exp(-jnp.abs(zr)))


def _head_call(root2, agg0, agg1, cntp, mu_pred, Whead, bhead, bconv2):
    return pl.pallas_call(
        _head_body,
        grid=(N // BN,),
        in_specs=[
            pl.BlockSpec((2, BN, H), lambda i: (0, i, 0)),
            pl.BlockSpec((2, BN, HH), lambda i: (0, i, 0)),
            pl.BlockSpec((2, BN, HH), lambda i: (0, i, 0)),
            pl.BlockSpec((2, BN, 16), lambda i: (0, i, 0)),
            pl.BlockSpec((2, BN, 1), lambda i: (0, i, 0)),
            pl.BlockSpec((H, 2), lambda i: (0, 0)),
            pl.BlockSpec((1, 2), lambda i: (0, 0)),
            pl.BlockSpec((1, H), lambda i: (0, 0)),
        ],
        out_specs=[
            pl.BlockSpec((2, BN, 1), lambda i: (0, i, 0)),
            pl.BlockSpec((2, BN, 1), lambda i: (0, i, 0)),
        ],
        out_shape=[
            jax.ShapeDtypeStruct((2, N, 1), _f32),
            jax.ShapeDtypeStruct((2, N, 1), _f32),
        ],
    )(root2, agg0, agg1, cntp, mu_pred, Whead, bhead, bconv2)


@functools.cache
def _mesh():
    return plsc.VectorSubcoreMesh(
        core_axis_name="c", subcore_axis_name="s", num_cores=2, num_subcores=16)


def _fill_zeros(zbuf, rows, cols):
    zero = jnp.zeros((16,), _f32)

    def body(i, _):
        for h in range(cols // 16):
            zbuf[i, pl.ds(h * 16, 16)] = zero
        return 0

    lax.fori_loop(0, rows, body, 0)


def _cnt_kernel(dst_hbm, out_hbm, dstg, ones, cnt_s, zbuf):
    c = lax.axis_index("c")
    s = lax.axis_index("s")

    _fill_zeros(zbuf, ZROWS, 16)
    one = jnp.full((16,), 1.0, _f32)

    def fill_ones(i, _):
        ones[i, pl.ds(0, 16)] = one
        return 0

    lax.fori_loop(0, CHUNK, fill_ones, 0)

    row0 = s * STRIPE
    for q in range(4):
        pltpu.sync_copy(zbuf, cnt_s.at[pl.ds(row0 + q * ZROWS, ZROWS), :])
    plsc.subcore_barrier()

    base = c * (NCH // 2) + s * CNT_CPT
    for g in range(CNT_CPT // GRP):
        pltpu.sync_copy(dst_hbm.at[pl.ds(base + g * GRP, GRP), :], dstg)

        def body(k, _):
            pltpu.sync_copy(ones, cnt_s.at[dstg.at[k]], add=True)
            return 0

        lax.fori_loop(0, GRP, body, 0)

    plsc.subcore_barrier()
    pltpu.sync_copy(cnt_s.at[pl.ds(row0, STRIPE), :],
                    out_hbm.at[c, pl.ds(row0, STRIPE), :])


def _cnt_call(dst_c):
    f = functools.partial(
        pl.kernel,
        out_type=jax.ShapeDtypeStruct((2, NPAD, 16), _f32),
        mesh=_mesh(),
        compiler_params=pltpu.CompilerParams(use_tc_tiling_on_sc=False),
        scratch_types=[
            pltpu.VMEM((GRP, CHUNK), _i32),
            pltpu.VMEM((CHUNK, 16), _f32),
            pltpu.VMEM_SHARED((NPAD, 16), _f32),
            pltpu.VMEM((ZROWS, 16), _f32),
        ],
    )
    return f(_cnt_kernel)(dst_c)


def _seg_kernel(tab0_hbm, tab1_hbm, gate_hbm, src_hbm, dst_hbm, zeros_hbm,
                out0_hbm, out1_hbm, srcg, dstg, rows0, rows1, gate0, gate1,
                agg_s, gsem0, gsem1, tsem0, tsem1, ssem0, ssem1):
    c = lax.axis_index("c")
    s = lax.axis_index("s")
    row0 = s * STRIPE
    rows_b = (rows0, rows1)
    gate_b = (gate0, gate1)
    gsems = (gsem0, gsem1)
    tsems = (tsem0, tsem1)
    ssems = (ssem0, ssem1)

    for b, tab_hbm, out_hbm in ((0, tab0_hbm, out0_hbm), (1, tab1_hbm, out1_hbm)):
        pltpu.sync_copy(zeros_hbm.at[pl.ds(row0, STRIPE), :],
                        agg_s.at[pl.ds(row0, STRIPE), :])
        plsc.subcore_barrier()

        base = s * CPT

        def group(g, _):
            g0 = base + g * SGRP
            pltpu.sync_copy(src_hbm.at[pl.ds(g0, SGRP), :], srcg)
            pltpu.sync_copy(dst_hbm.at[pl.ds(g0, SGRP), :], dstg)

            def issue(k):
                sl = k % 2
                gd = pltpu.async_copy(
                    tab_hbm.at[c].at[srcg.at[k]], rows_b[sl], gsems[sl])
                td = pltpu.async_copy(
                    gate_hbm.at[c, pl.ds((g0 + k) * CHUNK, CHUNK), :],
                    gate_b[sl], tsems[sl])
                return gd, td

            gd = [None, None]
            td = [None, None]
            sd = [None, None]
            gd[0], td[0] = issue(0)
            for k in range(SGRP):
                sl = k % 2
                nsl = (k + 1) % 2
                if k + 1 < SGRP:
                    if sd[nsl] is not None:
                        sd[nsl].wait()
                    gd[nsl], td[nsl] = issue(k + 1)
                gd[sl].wait()
                td[sl].wait()
                rows = rows_b[sl]
                gatev = gate_b[sl]

                @plsc.parallel_loop(0, CHUNK, unroll=8)
                def mul(r):
                    rows[r, pl.ds(0, 16)] = (
                        rows[r, pl.ds(0, 16)] * gatev[r, pl.ds(0, 16)])
                    rows[r, pl.ds(16, 16)] = (
                        rows[r, pl.ds(16, 16)] * gatev[r, pl.ds(16, 16)])

                sd[sl] = pltpu.async_copy(
                    rows, agg_s.at[dstg.at[k]], ssems[sl], add=True)
            sd[0].wait()
            sd[1].wait()
            return 0

        lax.fori_loop(0, CPT // SGRP, group, 0)

        plsc.subcore_barrier()
        pltpu.sync_copy(agg_s.at[pl.ds(row0, STRIPE), :],
                        out_hbm.at[c, pl.ds(row0, STRIPE), :])
        plsc.subcore_barrier()


def _seg_call(tab0, tab1, gate, src_c, dst_c, zeros):
    f = functools.partial(
        pl.kernel,
        out_type=[
            jax.ShapeDtypeStruct((2, NPAD, HH), _f32),
            jax.ShapeDtypeStruct((2, NPAD, HH), _f32),
        ],
        mesh=_mesh(),
        compiler_params=pltpu.CompilerParams(use_tc_tiling_on_sc=False),
        scratch_types=[
            pltpu.VMEM((SGRP, CHUNK), _i32),
            pltpu.VMEM((SGRP, CHUNK), _i32),
            pltpu.VMEM((CHUNK, HH), _f32),
            pltpu.VMEM((CHUNK, HH), _f32),
            pltpu.VMEM((CHUNK, HH), _f32),
            pltpu.VMEM((CHUNK, HH), _f32),
            pltpu.VMEM_SHARED((NPAD, HH), _f32),
            pltpu.SemaphoreType.DMA,
            pltpu.SemaphoreType.DMA,
            pltpu.SemaphoreType.DMA,
            pltpu.SemaphoreType.DMA,
            pltpu.SemaphoreType.DMA,
            pltpu.SemaphoreType.DMA,
        ],
    )
    return f(_seg_kernel)(tab0, tab1, gate, src_c, dst_c, zeros)


def kernel(mu_pred, sigma_pred, edge_index, edge_attr, pos_emb, We1, be1, We2,
           be2, Wroot1, Wmsg1, bconv1, Wroot2, Wmsg2, bconv2, Wmu, bmu, Wrad,
           brad):
    src = edge_index[0].astype(_i32)
    dst = edge_index[1].astype(_i32)
    src_c = jnp.reshape(
        jnp.concatenate([src, jnp.zeros((EP - E,), _i32)]), (NCH, CHUNK))
    dst_c = jnp.reshape(
        jnp.concatenate([dst, jnp.full((EP - E,), N, _i32)]), (NCH, CHUNK))

    gate = _gate_call(edge_attr, We1, be1[None, :], We2, be2[None, :])
    cntp = _cnt_call(dst_c)

    y1_b0, y1_b1, root1 = _node1_call(sigma_pred[..., None], pos_emb, Wmsg1, Wroot1)
    zeros = jnp.zeros((NPAD, HH), _f32)
    agg1_b0, agg1_b1 = _seg_call(y1_b0, y1_b1, gate, src_c, dst_c, zeros)

    y2_b0, y2_b1, root2 = _mid_call(
        root1, agg1_b0, agg1_b1, cntp, Wmsg2, Wroot2, bconv1[None, :])
    agg2_b0, agg2_b1 = _seg_call(y2_b0, y2_b1, gate, src_c, dst_c, zeros)

    Whead = jnp.concatenate([Wmu, Wrad], axis=1)
    bhead = jnp.concatenate([bmu, brad])[None, :]
    mu_out, r_out = _head_call(
        root2, agg2_b0, agg2_b1, cntp, mu_pred[..., None], Whead, bhead,
        bconv2[None, :])
    return (mu_out[..., 0], r_out[..., 0])


# speedup vs baseline: 3.9496x; 1.1419x over previous
import functools

import jax
import jax.numpy as jnp
from jax import lax
from jax.experimental import pallas as pl
from jax.experimental.pallas import tpu as pltpu
from jax.experimental.pallas import tpu_sc as plsc

N = 50000
E = 800000
H = 64
HH = 32

CHUNK = 128
EP = 819200
NCH = EP // CHUNK
CPT = NCH // 16
CNT_CPT = NCH // 32
GRP = 40
NPAD = 50048
STRIPE = NPAD // 16
ZROWS = STRIPE // 4
SGRP = 16
SZROWS = STRIPE // 8

BN = 2000
BE = 3200

_f32 = jnp.float32
_i32 = jnp.int32


def _gate_body(ea_ref, we1_ref, be1_ref, we2_ref, be2_ref, out_ref):
    h = jnp.maximum(
        jnp.dot(ea_ref[...], we1_ref[...], preferred_element_type=_f32)
        + be1_ref[...], 0.0)
    g = jnp.dot(h, we2_ref[...], preferred_element_type=_f32) + be2_ref[...]
    out_ref[0] = g[:, :HH]
    out_ref[1] = g[:, HH:]


def _gate_call(edge_attr, We1, be1, We2, be2):
    return pl.pallas_call(
        _gate_body,
        grid=(E // BE,),
        in_specs=[
            pl.BlockSpec((BE, 16), lambda i: (i, 0)),
            pl.BlockSpec((16, H), lambda i: (0, 0)),
            pl.BlockSpec((1, H), lambda i: (0, 0)),
            pl.BlockSpec((H, H), lambda i: (0, 0)),
            pl.BlockSpec((1, H), lambda i: (0, 0)),
        ],
        out_specs=pl.BlockSpec((2, BE, HH), lambda i: (0, i, 0)),
        out_shape=jax.ShapeDtypeStruct((2, EP, HH), _f32),
    )(edge_attr, We1, be1, We2, be2)


def _node1_body(sig_ref, pos_ref, wmsg_ref, wroot_ref, y0_ref, y1_ref, root_ref):
    pos = pos_ref[...]
    for b, y_ref in ((0, y0_ref), (1, y1_ref)):
        x = jnp.concatenate([sig_ref[b], pos], axis=1)
        y = jnp.dot(x, wmsg_ref[...], preferred_element_type=_f32)
        y_ref[0] = y[:, :HH]
        y_ref[1] = y[:, HH:]
        root_ref[b] = jnp.dot(x, wroot_ref[...], preferred_element_type=_f32)


def _node1_call(sigma, pos_emb, Wmsg1, Wroot1):
    return pl.pallas_call(
        _node1_body,
        grid=(N // BN,),
        in_specs=[
            pl.BlockSpec((2, BN, 1), lambda i: (0, i, 0)),
            pl.BlockSpec((BN, 16), lambda i: (i, 0)),
            pl.BlockSpec((17, H), lambda i: (0, 0)),
            pl.BlockSpec((17, H), lambda i: (0, 0)),
        ],
        out_specs=[
            pl.BlockSpec((2, BN, HH), lambda i: (0, i, 0)),
            pl.BlockSpec((2, BN, HH), lambda i: (0, i, 0)),
            pl.BlockSpec((2, BN, H), lambda i: (0, i, 0)),
        ],
        out_shape=[
            jax.ShapeDtypeStruct((2, N, HH), _f32),
            jax.ShapeDtypeStruct((2, N, HH), _f32),
            jax.ShapeDtypeStruct((2, N, H), _f32),
        ],
    )(sigma, pos_emb, Wmsg1, Wroot1)


def _rcnt_from(cntp):
    cnt = jnp.maximum(cntp[0, :, 0] + cntp[1, :, 0], 1.0)
    return (1.0 / cnt)[:, None]


def _mid_body(root1_ref, a_ref, cnt_ref, wmsg_ref, wroot_ref, b1_ref,
              y_ref, root2_ref):
    rcnt = _rcnt_from(cnt_ref[...])
    agg = jnp.concatenate([a_ref[0], a_ref[1]], axis=1) * rcnt
    h = jnp.maximum(root1_ref[0] + agg + b1_ref[...], 0.0)
    y = jnp.dot(h, wmsg_ref[...], preferred_element_type=_f32)
    y_ref[0] = y[:, :HH]
    y_ref[1] = y[:, HH:]
    root2_ref[...] = jnp.dot(h, wroot_ref[...], preferred_element_type=_f32)


def _mid_call(b, root1, agg, cntp, Wmsg2, Wroot2, bconv1):
    return pl.pallas_call(
        _mid_body,
        grid=(N // BN,),
        in_specs=[
            pl.BlockSpec((1, BN, H), lambda i, b=b: (b, i, 0)),
            pl.BlockSpec((2, BN, HH), lambda i: (0, i, 0)),
            pl.BlockSpec((2, BN, 16), lambda i: (0, i, 0)),
            pl.BlockSpec((H, H), lambda i: (0, 0)),
            pl.BlockSpec((H, H), lambda i: (0, 0)),
            pl.BlockSpec((1, H), lambda i: (0, 0)),
        ],
        out_specs=[
            pl.BlockSpec((2, BN, HH), lambda i: (0, i, 0)),
            pl.BlockSpec((BN, H), lambda i: (i, 0)),
        ],
        out_shape=[
            jax.ShapeDtypeStruct((2, N, HH), _f32),
            jax.ShapeDtypeStruct((N, H), _f32),
        ],
    )(root1, agg, cntp, Wmsg2, Wroot2, bconv1)


def _head_body(root2_ref, a_ref, cnt_ref, mu_ref, wh_ref, bh_ref, b2_ref,
               muo_ref, ro_ref):
    rcnt = _rcnt_from(cnt_ref[...])
    agg = jnp.concatenate([a_ref[0], a_ref[1]], axis=1) * rcnt
    h = jnp.maximum(root2_ref[...] + agg + b2_ref[...], 0.0)
    z = jnp.dot(h, wh_ref[...], preferred_element_type=_f32) + bh_ref[...]
    muo_ref[...] = mu_ref[0] + z[:, 0:1]
    zr = z[:, 1:2]
    ro_ref[...] = jnp.maximum(zr, 0.0) + jnp.log1p(jnp.exp(-jnp.abs(zr)))


def _head_call(b, root2, agg, cntp, mu_pred, Whead, bhead, bconv2):
    return pl.pallas_call(
        _head_body,
        grid=(N // BN,),
        in_specs=[
            pl.BlockSpec((BN, H), lambda i: (i, 0)),
            pl.BlockSpec((2, BN, HH), lambda i: (0, i, 0)),
            pl.BlockSpec((2, BN, 16), lambda i: (0, i, 0)),
            pl.BlockSpec((1, BN, 1), lambda i, b=b: (b, i, 0)),
            pl.BlockSpec((H, 2), lambda i: (0, 0)),
            pl.BlockSpec((1, 2), lambda i: (0, 0)),
            pl.BlockSpec((1, H), lambda i: (0, 0)),
        ],
        out_specs=[
            pl.BlockSpec((BN, 1), lambda i: (i, 0)),
            pl.BlockSpec((BN, 1), lambda i: (i, 0)),
        ],
        out_shape=[
            jax.ShapeDtypeStruct((N, 1), _f32),
            jax.ShapeDtypeStruct((N, 1), _f32),
        ],
    )(root2, agg, cntp, mu_pred, Whead, bhead, bconv2)


@functools.cache
def _mesh():
    return plsc.VectorSubcoreMesh(
        core_axis_name="c", subcore_axis_name="s", num_cores=2, num_subcores=16)


def _fill_zeros(zbuf, rows, cols):
    zero = jnp.zeros((16,), _f32)

    def body(i, _):
        for h in range(cols // 16):
            zbuf[i, pl.ds(h * 16, 16)] = zero
        return 0

    lax.fori_loop(0, rows, body, 0)


def _cnt_kernel(dst_hbm, out_hbm, dstg, ones, cnt_s, zbuf):
    c = lax.axis_index("c")
    s = lax.axis_index("s")

    _fill_zeros(zbuf, ZROWS, 16)
    one = jnp.full((16,), 1.0, _f32)

    def fill_ones(i, _):
        ones[i, pl.ds(0, 16)] = one
        return 0

    lax.fori_loop(0, CHUNK, fill_ones, 0)

    row0 = s * STRIPE
    for q in range(4):
        pltpu.sync_copy(zbuf, cnt_s.at[pl.ds(row0 + q * ZROWS, ZROWS), :])
    plsc.subcore_barrier()

    base = c * (NCH // 2) + s * CNT_CPT
    for g in range(CNT_CPT // GRP):
        pltpu.sync_copy(dst_hbm.at[pl.ds(base + g * GRP, GRP), :], dstg)

        def body(k, _):
            pltpu.sync_copy(ones, cnt_s.at[dstg.at[k]], add=True)
            return 0

        lax.fori_loop(0, GRP, body, 0)

    plsc.subcore_barrier()
    pltpu.sync_copy(cnt_s.at[pl.ds(row0, STRIPE), :],
                    out_hbm.at[c, pl.ds(row0, STRIPE), :])


def _cnt_call(dst_c):
    f = functools.partial(
        pl.kernel,
        out_type=jax.ShapeDtypeStruct((2, NPAD, 16), _f32),
        mesh=_mesh(),
        compiler_params=pltpu.CompilerParams(use_tc_tiling_on_sc=False),
        scratch_types=[
            pltpu.VMEM((GRP, CHUNK), _i32),
            pltpu.VMEM((CHUNK, 16), _f32),
            pltpu.VMEM_SHARED((NPAD, 16), _f32),
            pltpu.VMEM((ZROWS, 16), _f32),
        ],
    )
    return f(_cnt_kernel)(dst_c)


def _seg_kernel(tab_hbm, gate_hbm, src_hbm, dst_hbm, zeros_hbm, out_hbm,
                srcg, dstg, rows0, rows1, rows2, gate0, gate1, gate2, agg_s,
                gsem0, gsem1, gsem2, tsem0, tsem1, tsem2, ssem0, ssem1, ssem2):
    c = lax.axis_index("c")
    s = lax.axis_index("s")
    row0 = s * STRIPE
    rows_b = (rows0, rows1, rows2)
    gate_b = (gate0, gate1, gate2)
    gsems = (gsem0, gsem1, gsem2)
    tsems = (tsem0, tsem1, tsem2)
    ssems = (ssem0, ssem1, ssem2)

    pltpu.sync_copy(zeros_hbm.at[pl.ds(row0, STRIPE), :],
                    agg_s.at[pl.ds(row0, STRIPE), :])
    plsc.subcore_barrier()

    base = s * CPT

    def group(g, _):
        g0 = base + g * SGRP
        pltpu.sync_copy(src_hbm.at[pl.ds(g0, SGRP), :], srcg)
        pltpu.sync_copy(dst_hbm.at[pl.ds(g0, SGRP), :], dstg)

        def issue(k):
            sl = k % 3
            gd = pltpu.async_copy(
                tab_hbm.at[c].at[srcg.at[k]], rows_b[sl], gsems[sl])
            td = pltpu.async_copy(
                gate_hbm.at[c, pl.ds((g0 + k) * CHUNK, CHUNK), :],
                gate_b[sl], tsems[sl])
            return gd, td

        gd = [None, None, None]
        td = [None, None, None]
        sd = [None, None, None]
        gd[0], td[0] = issue(0)
        gd[1], td[1] = issue(1)
        for k in range(SGRP):
            sl = k % 3
            if k + 2 < SGRP:
                nsl = (k + 2) % 3
                if sd[nsl] is not None:
                    sd[nsl].wait()
                gd[nsl], td[nsl] = issue(k + 2)
            gd[sl].wait()
            td[sl].wait()
            rows = rows_b[sl]
            gatev = gate_b[sl]

            @plsc.parallel_loop(0, CHUNK, unroll=8)
            def mul(r):
                rows[r, pl.ds(0, 16)] = (
                    rows[r, pl.ds(0, 16)] * gatev[r, pl.ds(0, 16)])
                rows[r, pl.ds(16, 16)] = (
                    rows[r, pl.ds(16, 16)] * gatev[r, pl.ds(16, 16)])

            sd[sl] = pltpu.async_copy(
                rows, agg_s.at[dstg.at[k]], ssems[sl], add=True)
        sd[0].wait()
        sd[1].wait()
        sd[2].wait()
        return 0

    lax.fori_loop(0, CPT // SGRP, group, 0)

    plsc.subcore_barrier()
    pltpu.sync_copy(agg_s.at[pl.ds(row0, STRIPE), :],
                    out_hbm.at[c, pl.ds(row0, STRIPE), :])


def _seg_call(tab, gate, src_c, dst_c, zeros):
    f = functools.partial(
        pl.kernel,
        out_type=jax.ShapeDtypeStruct((2, NPAD, HH), _f32),
        mesh=_mesh(),
        compiler_params=pltpu.CompilerParams(use_tc_tiling_on_sc=False),
        scratch_types=(
            [pltpu.VMEM((SGRP, CHUNK), _i32)] * 2
            + [pltpu.VMEM((CHUNK, HH), _f32)] * 6
            + [pltpu.VMEM_SHARED((NPAD, HH), _f32)]
            + [pltpu.SemaphoreType.DMA] * 9
        ),
    )
    return f(_seg_kernel)(tab, gate, src_c, dst_c, zeros)


def kernel(mu_pred, sigma_pred, edge_index, edge_attr, pos_emb, We1, be1, We2,
           be2, Wroot1, Wmsg1, bconv1, Wroot2, Wmsg2, bconv2, Wmu, bmu, Wrad,
           brad):
    src = edge_index[0].astype(_i32)
    dst = edge_index[1].astype(_i32)
    src_c = jnp.reshape(
        jnp.concatenate([src, jnp.zeros((EP - E,), _i32)]), (NCH, CHUNK))
    dst_c = jnp.reshape(
        jnp.concatenate([dst, jnp.full((EP - E,), N, _i32)]), (NCH, CHUNK))

    gate = _gate_call(edge_attr, We1, be1[None, :], We2, be2[None, :])
    cntp = _cnt_call(dst_c)

    y1_b0, y1_b1, root1 = _node1_call(sigma_pred[..., None], pos_emb, Wmsg1,
                                      Wroot1)
    zeros = jnp.zeros((NPAD, HH), _f32)
    agg1_b0 = _seg_call(y1_b0, gate, src_c, dst_c, zeros)
    agg1_b1 = _seg_call(y1_b1, gate, src_c, dst_c, zeros)

    y2_b0, root2_b0 = _mid_call(0, root1, agg1_b0, cntp, Wmsg2, Wroot2,
                                bconv1[None, :])
    y2_b1, root2_b1 = _mid_call(1, root1, agg1_b1, cntp, Wmsg2, Wroot2,
                                bconv1[None, :])
    agg2_b0 = _seg_call(y2_b0, gate, src_c, dst_c, zeros)
    agg2_b1 = _seg_call(y2_b1, gate, src_c, dst_c, zeros)

    Whead = jnp.concatenate([Wmu, Wrad], axis=1)
    bhead = jnp.concatenate([bmu, brad])[None, :]
    mu3 = mu_pred[..., None]
    mu0, r0 = _head_call(0, root2_b0, agg2_b0, cntp, mu3, Whead, bhead,
                         bconv2[None, :])
    mu1, r1 = _head_call(1, root2_b1, agg2_b1, cntp, mu3, Whead, bhead,
                         bconv2[None, :])
    return (jnp.stack([mu0[:, 0], mu1[:, 0]]), jnp.stack([r0[:, 0], r1[:, 0]]))


# speedup vs baseline: 4.9312x; 1.2485x over previous
import functools

import jax
import jax.numpy as jnp
from jax import lax
from jax.experimental import pallas as pl
from jax.experimental.pallas import tpu as pltpu
from jax.experimental.pallas import tpu_sc as plsc

N = 50000
E = 800000
H = 64
HH = 32

CHUNK = 128
EP = 819200
NCH = EP // CHUNK
CPT = NCH // 16
CNT_CPT = NCH // 32
GRP = 40
NPAD = 50048
STRIPE = NPAD // 16
ZROWS = STRIPE // 4
SGRP = 16
SZROWS = STRIPE // 8

BN = 2000
BE = 3200
E4 = E // 4
EPQ = EP // 4

_f32 = jnp.float32
_i32 = jnp.int32


def _gate_body(ea_ref, w1_ref, b1_ref, w2_ref, b2_ref, out_ref):
    h = jnp.maximum(
        jnp.dot(ea_ref[...], w1_ref[...], preferred_element_type=_f32)
        + b1_ref[...], 0.0)
    g = jnp.dot(h, w2_ref[...], preferred_element_type=_f32) + b2_ref[...]
    out_ref[0] = g[:, :128]
    out_ref[1] = g[:, 128:]


def _gate_call(ea4, W1b, b1b, W2b, b2b):
    bq = BE // 4
    return pl.pallas_call(
        _gate_body,
        grid=(E4 // bq,),
        in_specs=[
            pl.BlockSpec((bq, 64), lambda i: (i, 0)),
            pl.BlockSpec((64, 256), lambda i: (0, 0)),
            pl.BlockSpec((1, 256), lambda i: (0, 0)),
            pl.BlockSpec((256, 256), lambda i: (0, 0)),
            pl.BlockSpec((1, 256), lambda i: (0, 0)),
        ],
        out_specs=pl.BlockSpec((2, bq, 128), lambda i: (0, i, 0)),
        out_shape=jax.ShapeDtypeStruct((2, EPQ, 128), _f32),
    )(ea4, W1b, b1b, W2b, b2b)


def _node1_body(sig_ref, pos_ref, wmsg_ref, wroot_ref, y0_ref, y1_ref, root_ref):
    pos = pos_ref[...]
    for b, y_ref in ((0, y0_ref), (1, y1_ref)):
        x = jnp.concatenate([sig_ref[b], pos], axis=1)
        y = jnp.dot(x, wmsg_ref[...], preferred_element_type=_f32)
        y_ref[0] = y[:, :HH]
        y_ref[1] = y[:, HH:]
        root_ref[b] = jnp.dot(x, wroot_ref[...], preferred_element_type=_f32)


def _node1_call(sigma, pos_emb, Wmsg1, Wroot1):
    return pl.pallas_call(
        _node1_body,
        grid=(N // BN,),
        in_specs=[
            pl.BlockSpec((2, BN, 1), lambda i: (0, i, 0)),
            pl.BlockSpec((BN, 16), lambda i: (i, 0)),
            pl.BlockSpec((17, H), lambda i: (0, 0)),
            pl.BlockSpec((17, H), lambda i: (0, 0)),
        ],
        out_specs=[
            pl.BlockSpec((2, BN, HH), lambda i: (0, i, 0)),
            pl.BlockSpec((2, BN, HH), lambda i: (0, i, 0)),
            pl.BlockSpec((2, BN, H), lambda i: (0, i, 0)),
        ],
        out_shape=[
            jax.ShapeDtypeStruct((2, N, HH), _f32),
            jax.ShapeDtypeStruct((2, N, HH), _f32),
            jax.ShapeDtypeStruct((2, N, H), _f32),
        ],
    )(sigma, pos_emb, Wmsg1, Wroot1)


def _rcnt_from(cntp):
    cnt = jnp.maximum(cntp[0, :, 0] + cntp[1, :, 0], 1.0)
    return (1.0 / cnt)[:, None]


def _mid_body(root1_ref, a_ref, cnt_ref, wmsg_ref, wroot_ref, b1_ref,
              y_ref, root2_ref):
    rcnt = _rcnt_from(cnt_ref[...])
    agg = jnp.concatenate([a_ref[0], a_ref[1]], axis=1) * rcnt
    h = jnp.maximum(root1_ref[0] + agg + b1_ref[...], 0.0)
    y = jnp.dot(h, wmsg_ref[...], preferred_element_type=_f32)
    y_ref[0] = y[:, :HH]
    y_ref[1] = y[:, HH:]
    root2_ref[...] = jnp.dot(h, wroot_ref[...], preferred_element_type=_f32)


def _mid_call(b, root1, agg, cntp, Wmsg2, Wroot2, bconv1):
    return pl.pallas_call(
        _mid_body,
        grid=(N // BN,),
        in_specs=[
            pl.BlockSpec((1, BN, H), lambda i, b=b: (b, i, 0)),
            pl.BlockSpec((2, BN, HH), lambda i: (0, i, 0)),
            pl.BlockSpec((2, BN, 16), lambda i: (0, i, 0)),
            pl.BlockSpec((H, H), lambda i: (0, 0)),
            pl.BlockSpec((H, H), lambda i: (0, 0)),
            pl.BlockSpec((1, H), lambda i: (0, 0)),
        ],
        out_specs=[
            pl.BlockSpec((2, BN, HH), lambda i: (0, i, 0)),
            pl.BlockSpec((BN, H), lambda i: (i, 0)),
        ],
        out_shape=[
            jax.ShapeDtypeStruct((2, N, HH), _f32),
            jax.ShapeDtypeStruct((N, H), _f32),
        ],
    )(root1, agg, cntp, Wmsg2, Wroot2, bconv1)


def _head_body(root2_ref, a_ref, cnt_ref, mu_ref, wh_ref, bh_ref, b2_ref,
               muo_ref, ro_ref):
    rcnt = _rcnt_from(cnt_ref[...])
    agg = jnp.concatenate([a_ref[0], a_ref[1]], axis=1) * rcnt
    h = jnp.maximum(root2_ref[...] + agg + b2_ref[...], 0.0)
    z = jnp.dot(h, wh_ref[...], preferred_element_type=_f32) + bh_ref[...]
    muo_ref[...] = mu_ref[0] + z[:, 0:1]
    zr = z[:, 1:2]
    ro_ref[...] = jnp.maximum(zr, 0.0) + jnp.log1p(jnp.exp(-jnp.abs(zr)))


def _head_call(b, root2, agg, cntp, mu_pred, Whead, bhead, bconv2):
    return pl.pallas_call(
        _head_body,
        grid=(N // BN,),
        in_specs=[
            pl.BlockSpec((BN, H), lambda i: (i, 0)),
            pl.BlockSpec((2, BN, HH), lambda i: (0, i, 0)),
            pl.BlockSpec((2, BN, 16), lambda i: (0, i, 0)),
            pl.BlockSpec((1, BN, 1), lambda i, b=b: (b, i, 0)),
            pl.BlockSpec((H, 2), lambda i: (0, 0)),
            pl.BlockSpec((1, 2), lambda i: (0, 0)),
            pl.BlockSpec((1, H), lambda i: (0, 0)),
        ],
        out_specs=[
            pl.BlockSpec((BN, 1), lambda i: (i, 0)),
            pl.BlockSpec((BN, 1), lambda i: (i, 0)),
        ],
        out_shape=[
            jax.ShapeDtypeStruct((N, 1), _f32),
            jax.ShapeDtypeStruct((N, 1), _f32),
        ],
    )(root2, agg, cntp, mu_pred, Whead, bhead, bconv2)


@functools.cache
def _mesh():
    return plsc.VectorSubcoreMesh(
        core_axis_name="c", subcore_axis_name="s", num_cores=2, num_subcores=16)


def _fill_zeros(zbuf, rows, cols):
    zero = jnp.zeros((16,), _f32)

    def body(i, _):
        for h in range(cols // 16):
            zbuf[i, pl.ds(h * 16, 16)] = zero
        return 0

    lax.fori_loop(0, rows, body, 0)


def _cnt_kernel(dst_hbm, out_hbm, dstg, ones, cnt_s, zbuf):
    c = lax.axis_index("c")
    s = lax.axis_index("s")

    _fill_zeros(zbuf, ZROWS, 16)
    one = jnp.full((16,), 1.0, _f32)

    def fill_ones(i, _):
        ones[i, pl.ds(0, 16)] = one
        return 0

    lax.fori_loop(0, CHUNK, fill_ones, 0)

    row0 = s * STRIPE
    for q in range(4):
        pltpu.sync_copy(zbuf, cnt_s.at[pl.ds(row0 + q * ZROWS, ZROWS), :])
    plsc.subcore_barrier()

    base = c * (NCH // 2) + s * CNT_CPT
    for g in range(CNT_CPT // GRP):
        pltpu.sync_copy(dst_hbm.at[pl.ds(base + g * GRP, GRP), :], dstg)

        def body(k, _):
            pltpu.sync_copy(ones, cnt_s.at[dstg.at[k]], add=True)
            return 0

        lax.fori_loop(0, GRP, body, 0)

    plsc.subcore_barrier()
    pltpu.sync_copy(cnt_s.at[pl.ds(row0, STRIPE), :],
                    out_hbm.at[c, pl.ds(row0, STRIPE), :])


def _cnt_call(dst_c):
    f = functools.partial(
        pl.kernel,
        out_type=jax.ShapeDtypeStruct((2, NPAD, 16), _f32),
        mesh=_mesh(),
        compiler_params=pltpu.CompilerParams(use_tc_tiling_on_sc=False),
        scratch_types=[
            pltpu.VMEM((GRP, CHUNK), _i32),
            pltpu.VMEM((CHUNK, 16), _f32),
            pltpu.VMEM_SHARED((NPAD, 16), _f32),
            pltpu.VMEM((ZROWS, 16), _f32),
        ],
    )
    return f(_cnt_kernel)(dst_c)


def _seg_kernel(tab_hbm, gate_hbm, src_hbm, dst_hbm, zeros_hbm, out_hbm,
                srcg, dstg, rows0, rows1, rows2, gate0, gate1, gate2, agg_s,
                gsem0, gsem1, gsem2, tsem0, tsem1, tsem2, ssem0, ssem1, ssem2):
    c = lax.axis_index("c")
    s = lax.axis_index("s")
    row0 = s * STRIPE
    rows_b = (rows0, rows1, rows2)
    gate_b = (gate0, gate1, gate2)
    gsems = (gsem0, gsem1, gsem2)
    tsems = (tsem0, tsem1, tsem2)
    ssems = (ssem0, ssem1, ssem2)

    pltpu.sync_copy(zeros_hbm.at[pl.ds(row0, STRIPE), :],
                    agg_s.at[pl.ds(row0, STRIPE), :])
    plsc.subcore_barrier()

    base = s * CPT

    def group(g, _):
        g0 = base + g * SGRP
        pltpu.sync_copy(src_hbm.at[pl.ds(g0, SGRP), :], srcg)
        pltpu.sync_copy(dst_hbm.at[pl.ds(g0, SGRP), :], dstg)

        def issue(k):
            sl = k % 3
            gd = pltpu.async_copy(
                tab_hbm.at[c].at[srcg.at[k]], rows_b[sl], gsems[sl])
            td = pltpu.async_copy(
                gate_hbm.at[c, pl.ds((g0 + k) * (CHUNK // 4), CHUNK // 4), :],
                gate_b[sl], tsems[sl])
            return gd, td

        gd = [None, None, None]
        td = [None, None, None]
        sd = [None, None, None]
        gd[0], td[0] = issue(0)
        gd[1], td[1] = issue(1)
        for k in range(SGRP):
            sl = k % 3
            if k + 2 < SGRP:
                nsl = (k + 2) % 3
                if sd[nsl] is not None:
                    sd[nsl].wait()
                gd[nsl], td[nsl] = issue(k + 2)
            gd[sl].wait()
            td[sl].wait()
            rows = rows_b[sl]
            gatev = gate_b[sl]

            @plsc.parallel_loop(0, CHUNK, unroll=8)
            def mul(r):
                rq = r // 4
                ro = (r % 4) * HH
                rows[r, pl.ds(0, 16)] = (
                    rows[r, pl.ds(0, 16)] * gatev[rq, pl.ds(ro, 16)])
                rows[r, pl.ds(16, 16)] = (
                    rows[r, pl.ds(16, 16)] * gatev[rq, pl.ds(ro + 16, 16)])

            sd[sl] = pltpu.async_copy(
                rows, agg_s.at[dstg.at[k]], ssems[sl], add=True)
        sd[0].wait()
        sd[1].wait()
        sd[2].wait()
        return 0

    lax.fori_loop(0, CPT // SGRP, group, 0)

    plsc.subcore_barrier()
    pltpu.sync_copy(agg_s.at[pl.ds(row0, STRIPE), :],
                    out_hbm.at[c, pl.ds(row0, STRIPE), :])


def _seg_call(tab, gate, src_c, dst_c, zeros):
    f = functools.partial(
        pl.kernel,
        out_type=jax.ShapeDtypeStruct((2, NPAD, HH), _f32),
        mesh=_mesh(),
        compiler_params=pltpu.CompilerParams(use_tc_tiling_on_sc=False),
        scratch_types=(
            [pltpu.VMEM((SGRP, CHUNK), _i32)] * 2
            + [pltpu.VMEM((CHUNK, HH), _f32)] * 3
            + [pltpu.VMEM((CHUNK // 4, 128), _f32)] * 3
            + [pltpu.VMEM_SHARED((NPAD, HH), _f32)]
            + [pltpu.SemaphoreType.DMA] * 9
        ),
    )
    return f(_seg_kernel)(tab, gate, src_c, dst_c, zeros)


def kernel(mu_pred, sigma_pred, edge_index, edge_attr, pos_emb, We1, be1, We2,
           be2, Wroot1, Wmsg1, bconv1, Wroot2, Wmsg2, bconv2, Wmu, bmu, Wrad,
           brad):
    src = edge_index[0].astype(_i32)
    dst = edge_index[1].astype(_i32)
    src_c = jnp.reshape(
        jnp.concatenate([src, jnp.zeros((EP - E,), _i32)]), (NCH, CHUNK))
    dst_c = jnp.reshape(
        jnp.concatenate([dst, jnp.full((EP - E,), N, _i32)]), (NCH, CHUNK))

    ea4 = jnp.reshape(edge_attr, (E4, 64))
    eye4 = jnp.eye(4, dtype=_f32)
    W1b = jnp.kron(eye4, We1)
    b1b = jnp.tile(be1, 4)[None, :]
    W2b = jnp.concatenate(
        [jnp.kron(eye4, We2[:, :HH]), jnp.kron(eye4, We2[:, HH:])], axis=1)
    b2b = jnp.concatenate(
        [jnp.tile(be2[:HH], 4), jnp.tile(be2[HH:], 4)])[None, :]
    gate = _gate_call(ea4, W1b, b1b, W2b, b2b)
    cntp = _cnt_call(dst_c)

    y1_b0, y1_b1, root1 = _node1_call(sigma_pred[..., None], pos_emb, Wmsg1,
                                      Wroot1)
    zeros = jnp.zeros((NPAD, HH), _f32)
    agg1_b0 = _seg_call(y1_b0, gate, src_c, dst_c, zeros)
    agg1_b1 = _seg_call(y1_b1, gate, src_c, dst_c, zeros)

    y2_b0, root2_b0 = _mid_call(0, root1, agg1_b0, cntp, Wmsg2, Wroot2,
                                bconv1[None, :])
    y2_b1, root2_b1 = _mid_call(1, root1, agg1_b1, cntp, Wmsg2, Wroot2,
                                bconv1[None, :])
    agg2_b0 = _seg_call(y2_b0, gate, src_c, dst_c, zeros)
    agg2_b1 = _seg_call(y2_b1, gate, src_c, dst_c, zeros)

    Whead = jnp.concatenate([Wmu, Wrad], axis=1)
    bhead = jnp.concatenate([bmu, brad])[None, :]
    mu3 = mu_pred[..., None]
    mu0, r0 = _head_call(0, root2_b0, agg2_b0, cntp, mu3, Whead, bhead,
                         bconv2[None, :])
    mu1, r1 = _head_call(1, root2_b1, agg2_b1, cntp, mu3, Whead, bhead,
                         bconv2[None, :])
    return (jnp.stack([mu0[:, 0], mu1[:, 0]]), jnp.stack([r0[:, 0], r1[:, 0]]))


# speedup vs baseline: 5.0251x; 1.0190x over previous
import functools

import jax
import jax.numpy as jnp
from jax import lax
from jax.experimental import pallas as pl
from jax.experimental.pallas import tpu as pltpu
from jax.experimental.pallas import tpu_sc as plsc

N = 50000
E = 800000
H = 64
HH = 32

CHUNK = 128
EP = 819200
NCH = EP // CHUNK
CPT = NCH // 16
CNT_CPT = NCH // 32
GRP = 40
NPAD = 50048
STRIPE = NPAD // 16
ZROWS = STRIPE // 4
SGRP = 16
SZROWS = STRIPE // 8

BN = 2000
BE = 3200
E4 = E // 4
EPQ = EP // 4

_f32 = jnp.float32
_i32 = jnp.int32


def _gate_body(ea_ref, w1_ref, b1_ref, w2_ref, b2_ref, out_ref):
    h = jnp.maximum(
        jnp.dot(ea_ref[...], w1_ref[...], preferred_element_type=_f32)
        + b1_ref[...], 0.0)
    g = jnp.dot(h, w2_ref[...], preferred_element_type=_f32) + b2_ref[...]
    out_ref[0] = g[:, :128]
    out_ref[1] = g[:, 128:]


def _gate_call(ea4, W1b, b1b, W2b, b2b):
    bq = BE // 4
    return pl.pallas_call(
        _gate_body,
        grid=(E4 // bq,),
        in_specs=[
            pl.BlockSpec((bq, 64), lambda i: (i, 0)),
            pl.BlockSpec((64, 256), lambda i: (0, 0)),
            pl.BlockSpec((1, 256), lambda i: (0, 0)),
            pl.BlockSpec((256, 256), lambda i: (0, 0)),
            pl.BlockSpec((1, 256), lambda i: (0, 0)),
        ],
        out_specs=pl.BlockSpec((2, bq, 128), lambda i: (0, i, 0)),
        out_shape=jax.ShapeDtypeStruct((2, EPQ, 128), _f32),
    )(ea4, W1b, b1b, W2b, b2b)


def _node1_body(sig_ref, pos_ref, wmsg_ref, wroot_ref, y0_ref, y1_ref, root_ref):
    pos = pos_ref[...]
    for b, y_ref in ((0, y0_ref), (1, y1_ref)):
        x = jnp.concatenate([sig_ref[b], pos], axis=1)
        y = jnp.dot(x, wmsg_ref[...], preferred_element_type=_f32)
        y_ref[0] = y[:, :HH]
        y_ref[1] = y[:, HH:]
        root_ref[b] = jnp.dot(x, wroot_ref[...], preferred_element_type=_f32)


def _node1_call(sigma, pos_emb, Wmsg1, Wroot1):
    return pl.pallas_call(
        _node1_body,
        grid=(N // BN,),
        in_specs=[
            pl.BlockSpec((2, BN, 1), lambda i: (0, i, 0)),
            pl.BlockSpec((BN, 16), lambda i: (i, 0)),
            pl.BlockSpec((17, H), lambda i: (0, 0)),
            pl.BlockSpec((17, H), lambda i: (0, 0)),
        ],
        out_specs=[
            pl.BlockSpec((2, BN, HH), lambda i: (0, i, 0)),
            pl.BlockSpec((2, BN, HH), lambda i: (0, i, 0)),
            pl.BlockSpec((2, BN, H), lambda i: (0, i, 0)),
        ],
        out_shape=[
            jax.ShapeDtypeStruct((2, N, HH), _f32),
            jax.ShapeDtypeStruct((2, N, HH), _f32),
            jax.ShapeDtypeStruct((2, N, H), _f32),
        ],
    )(sigma, pos_emb, Wmsg1, Wroot1)


def _rcnt_from(cntp):
    cnt = jnp.maximum(cntp[0, :, 0] + cntp[1, :, 0], 1.0)
    return (1.0 / cnt)[:, None]


def _mid_body(root1_ref, a_ref, cnt_ref, wmsg_ref, wroot_ref, b1_ref,
              y_ref, root2_ref):
    rcnt = _rcnt_from(cnt_ref[...])
    agg = jnp.concatenate([a_ref[0], a_ref[1]], axis=1) * rcnt
    h = jnp.maximum(root1_ref[0] + agg + b1_ref[...], 0.0)
    y = jnp.dot(h, wmsg_ref[...], preferred_element_type=_f32)
    y_ref[0] = y[:, :HH]
    y_ref[1] = y[:, HH:]
    root2_ref[...] = jnp.dot(h, wroot_ref[...], preferred_element_type=_f32)


def _mid_call(b, root1, agg, cntp, Wmsg2, Wroot2, bconv1):
    return pl.pallas_call(
        _mid_body,
        grid=(N // BN,),
        in_specs=[
            pl.BlockSpec((1, BN, H), lambda i, b=b: (b, i, 0)),
            pl.BlockSpec((2, BN, HH), lambda i: (0, i, 0)),
            pl.BlockSpec((2, BN, 16), lambda i: (0, i, 0)),
            pl.BlockSpec((H, H), lambda i: (0, 0)),
            pl.BlockSpec((H, H), lambda i: (0, 0)),
            pl.BlockSpec((1, H), lambda i: (0, 0)),
        ],
        out_specs=[
            pl.BlockSpec((2, BN, HH), lambda i: (0, i, 0)),
            pl.BlockSpec((BN, H), lambda i: (i, 0)),
        ],
        out_shape=[
            jax.ShapeDtypeStruct((2, N, HH), _f32),
            jax.ShapeDtypeStruct((N, H), _f32),
        ],
    )(root1, agg, cntp, Wmsg2, Wroot2, bconv1)


def _head_body(root2_ref, a_ref, cnt_ref, mu_ref, wh_ref, bh_ref, b2_ref,
               muo_ref, ro_ref):
    rcnt = _rcnt_from(cnt_ref[...])
    agg = jnp.concatenate([a_ref[0], a_ref[1]], axis=1) * rcnt
    h = jnp.maximum(root2_ref[...] + agg + b2_ref[...], 0.0)
    z = jnp.dot(h, wh_ref[...], preferred_element_type=_f32) + bh_ref[...]
    muo_ref[...] = mu_ref[0] + z[:, 0:1]
    zr = z[:, 1:2]
    ro_ref[...] = jnp.maximum(zr, 0.0) + jnp.log1p(jnp.exp(-jnp.abs(zr)))


def _head_call(b, root2, agg, cntp, mu_pred, Whead, bhead, bconv2):
    return pl.pallas_call(
        _head_body,
        grid=(N // BN,),
        in_specs=[
            pl.BlockSpec((BN, H), lambda i: (i, 0)),
            pl.BlockSpec((2, BN, HH), lambda i: (0, i, 0)),
            pl.BlockSpec((2, BN, 16), lambda i: (0, i, 0)),
            pl.BlockSpec((1, BN, 1), lambda i, b=b: (b, i, 0)),
            pl.BlockSpec((H, 2), lambda i: (0, 0)),
            pl.BlockSpec((1, 2), lambda i: (0, 0)),
            pl.BlockSpec((1, H), lambda i: (0, 0)),
        ],
        out_specs=[
            pl.BlockSpec((BN, 1), lambda i: (i, 0)),
            pl.BlockSpec((BN, 1), lambda i: (i, 0)),
        ],
        out_shape=[
            jax.ShapeDtypeStruct((N, 1), _f32),
            jax.ShapeDtypeStruct((N, 1), _f32),
        ],
    )(root2, agg, cntp, mu_pred, Whead, bhead, bconv2)


@functools.cache
def _mesh():
    return plsc.VectorSubcoreMesh(
        core_axis_name="c", subcore_axis_name="s", num_cores=2, num_subcores=16)


def _fill_zeros(zbuf, rows, cols):
    zero = jnp.zeros((16,), _f32)

    def body(i, _):
        for h in range(cols // 16):
            zbuf[i, pl.ds(h * 16, 16)] = zero
        return 0

    lax.fori_loop(0, rows, body, 0)


def _cnt_kernel(dst_hbm, out_hbm, dstg, ones, cnt_s, zbuf):
    c = lax.axis_index("c")
    s = lax.axis_index("s")

    _fill_zeros(zbuf, ZROWS, 16)
    one = jnp.full((16,), 1.0, _f32)

    def fill_ones(i, _):
        ones[i, pl.ds(0, 16)] = one
        return 0

    lax.fori_loop(0, CHUNK, fill_ones, 0)

    row0 = s * STRIPE
    for q in range(4):
        pltpu.sync_copy(zbuf, cnt_s.at[pl.ds(row0 + q * ZROWS, ZROWS), :])
    plsc.subcore_barrier()

    base = c * (NCH // 2) + s * CNT_CPT
    for g in range(CNT_CPT // GRP):
        pltpu.sync_copy(dst_hbm.at[pl.ds(base + g * GRP, GRP), :], dstg)

        def body(k, _):
            pltpu.sync_copy(ones, cnt_s.at[dstg.at[k]], add=True)
            return 0

        lax.fori_loop(0, GRP, body, 0)

    plsc.subcore_barrier()
    pltpu.sync_copy(cnt_s.at[pl.ds(row0, STRIPE), :],
                    out_hbm.at[c, pl.ds(row0, STRIPE), :])


def _cnt_call(dst_c):
    f = functools.partial(
        pl.kernel,
        out_type=jax.ShapeDtypeStruct((2, NPAD, 16), _f32),
        mesh=_mesh(),
        compiler_params=pltpu.CompilerParams(use_tc_tiling_on_sc=False),
        scratch_types=[
            pltpu.VMEM((GRP, CHUNK), _i32),
            pltpu.VMEM((CHUNK, 16), _f32),
            pltpu.VMEM_SHARED((NPAD, 16), _f32),
            pltpu.VMEM((ZROWS, 16), _f32),
        ],
    )
    return f(_cnt_kernel)(dst_c)


def _seg_kernel(tab_hbm, gate_hbm, src_hbm, dst_hbm, zeros_hbm, out_hbm,
                srcg, dstg, rows0, rows1, rows2, gate0, gate1, gate2, agg_s,
                gsem0, gsem1, gsem2, tsem0, tsem1, tsem2, ssem0, ssem1, ssem2):
    c = lax.axis_index("c")
    s = lax.axis_index("s")
    row0 = s * STRIPE
    rows_b = (rows0, rows1, rows2)
    gate_b = (gate0, gate1, gate2)
    gsems = (gsem0, gsem1, gsem2)
    tsems = (tsem0, tsem1, tsem2)
    ssems = (ssem0, ssem1, ssem2)

    pltpu.sync_copy(zeros_hbm.at[pl.ds(row0, STRIPE), :],
                    agg_s.at[pl.ds(row0, STRIPE), :])
    plsc.subcore_barrier()

    base = s * CPT

    def group(g, _):
        g0 = base + g * SGRP
        i1 = pltpu.async_copy(src_hbm.at[pl.ds(g0, SGRP), :], srcg, gsem0)
        i2 = pltpu.async_copy(dst_hbm.at[pl.ds(g0, SGRP), :], dstg, gsem0)

        @pl.when(g > 0)
        def _():
            for sl in range(3):
                pltpu.make_async_copy(
                    rows_b[sl], agg_s.at[dstg.at[0]], ssems[sl]).wait()

        i1.wait()
        i2.wait()

        def issue(k):
            sl = k % 3
            gd = pltpu.async_copy(
                tab_hbm.at[c].at[srcg.at[k]], rows_b[sl], gsems[sl])
            td = pltpu.async_copy(
                gate_hbm.at[c, pl.ds((g0 + k) * (CHUNK // 4), CHUNK // 4), :],
                gate_b[sl], tsems[sl])
            return gd, td

        gd = [None, None, None]
        td = [None, None, None]
        sd = [None, None, None]
        gd[0], td[0] = issue(0)
        gd[1], td[1] = issue(1)
        for k in range(SGRP):
            sl = k % 3
            if k + 2 < SGRP:
                nsl = (k + 2) % 3
                if sd[nsl] is not None:
                    sd[nsl].wait()
                gd[nsl], td[nsl] = issue(k + 2)
            gd[sl].wait()
            td[sl].wait()
            rows = rows_b[sl]
            gatev = gate_b[sl]

            @plsc.parallel_loop(0, CHUNK, unroll=8)
            def mul(r):
                rq = r // 4
                ro = (r % 4) * HH
                rows[r, pl.ds(0, 16)] = (
                    rows[r, pl.ds(0, 16)] * gatev[rq, pl.ds(ro, 16)])
                rows[r, pl.ds(16, 16)] = (
                    rows[r, pl.ds(16, 16)] * gatev[rq, pl.ds(ro + 16, 16)])

            sd[sl] = pltpu.async_copy(
                rows, agg_s.at[dstg.at[k]], ssems[sl], add=True)
        return 0

    lax.fori_loop(0, CPT // SGRP, group, 0)
    for sl in range(3):
        pltpu.make_async_copy(
            rows_b[sl], agg_s.at[dstg.at[0]], ssems[sl]).wait()

    plsc.subcore_barrier()
    pltpu.sync_copy(agg_s.at[pl.ds(row0, STRIPE), :],
                    out_hbm.at[c, pl.ds(row0, STRIPE), :])


def _seg_call(tab, gate, src_c, dst_c, zeros):
    f = functools.partial(
        pl.kernel,
        out_type=jax.ShapeDtypeStruct((2, NPAD, HH), _f32),
        mesh=_mesh(),
        compiler_params=pltpu.CompilerParams(use_tc_tiling_on_sc=False),
        scratch_types=(
            [pltpu.VMEM((SGRP, CHUNK), _i32)] * 2
            + [pltpu.VMEM((CHUNK, HH), _f32)] * 3
            + [pltpu.VMEM((CHUNK // 4, 128), _f32)] * 3
            + [pltpu.VMEM_SHARED((NPAD, HH), _f32)]
            + [pltpu.SemaphoreType.DMA] * 9
        ),
    )
    return f(_seg_kernel)(tab, gate, src_c, dst_c, zeros)


def kernel(mu_pred, sigma_pred, edge_index, edge_attr, pos_emb, We1, be1, We2,
           be2, Wroot1, Wmsg1, bconv1, Wroot2, Wmsg2, bconv2, Wmu, bmu, Wrad,
           brad):
    src = edge_index[0].astype(_i32)
    dst = edge_index[1].astype(_i32)
    src_c = jnp.reshape(
        jnp.concatenate([src, jnp.zeros((EP - E,), _i32)]), (NCH, CHUNK))
    dst_c = jnp.reshape(
        jnp.concatenate([dst, jnp.full((EP - E,), N, _i32)]), (NCH, CHUNK))

    ea4 = jnp.reshape(edge_attr, (E4, 64))
    eye4 = jnp.eye(4, dtype=_f32)
    W1b = jnp.kron(eye4, We1)
    b1b = jnp.tile(be1, 4)[None, :]
    W2b = jnp.concatenate(
        [jnp.kron(eye4, We2[:, :HH]), jnp.kron(eye4, We2[:, HH:])], axis=1)
    b2b = jnp.concatenate(
        [jnp.tile(be2[:HH], 4), jnp.tile(be2[HH:], 4)])[None, :]
    gate = _gate_call(ea4, W1b, b1b, W2b, b2b)
    cntp = _cnt_call(dst_c)

    y1_b0, y1_b1, root1 = _node1_call(sigma_pred[..., None], pos_emb, Wmsg1,
                                      Wroot1)
    zeros = jnp.zeros((NPAD, HH), _f32)
    agg1_b0 = _seg_call(y1_b0, gate, src_c, dst_c, zeros)
    agg1_b1 = _seg_call(y1_b1, gate, src_c, dst_c, zeros)

    y2_b0, root2_b0 = _mid_call(0, root1, agg1_b0, cntp, Wmsg2, Wroot2,
                                bconv1[None, :])
    y2_b1, root2_b1 = _mid_call(1, root1, agg1_b1, cntp, Wmsg2, Wroot2,
                                bconv1[None, :])
    agg2_b0 = _seg_call(y2_b0, gate, src_c, dst_c, zeros)
    agg2_b1 = _seg_call(y2_b1, gate, src_c, dst_c, zeros)

    Whead = jnp.concatenate([Wmu, Wrad], axis=1)
    bhead = jnp.concatenate([bmu, brad])[None, :]
    mu3 = mu_pred[..., None]
    mu0, r0 = _head_call(0, root2_b0, agg2_b0, cntp, mu3, Whead, bhead,
                         bconv2[None, :])
    mu1, r1 = _head_call(1, root2_b1, agg2_b1, cntp, mu3, Whead, bhead,
                         bconv2[None, :])
    return (jnp.stack([mu0[:, 0], mu1[:, 0]]), jnp.stack([r0[:, 0], r1[:, 0]]))
